# Initial kernel scaffold; baseline (speedup 1.0000x reference)
import jax
import jax.numpy as jnp
from jax.experimental import pallas as pl


def kernel(center, context, center_weight, center_bias, context_weight, context_bias):
    raise NotImplementedError("write your pallas kernel here")


# speedup vs baseline: 8.7501x; 8.7501x over previous
import functools

import jax
import jax.numpy as jnp
from jax import lax
from jax.experimental import pallas as pl
from jax.experimental.pallas import tpu as pltpu
from jax.experimental.pallas import tpu_sc as plsc

_INFO = plsc.get_sparse_core_info()
_NC = _INFO.num_cores
_NS = _INFO.num_subcores
_L = _INFO.num_lanes
_NW = _NC * _NS


def _make_glove_kernel(B, V, D, VP):
  BW = B // _NW
  NCH = BW // 128
  NG = 128 // _L

  mesh = plsc.VectorSubcoreMesh(core_axis_name="c", subcore_axis_name="s")

  @functools.partial(
      pl.kernel,
      mesh=mesh,
      out_type=jax.ShapeDtypeStruct((B,), jnp.float32),
      compiler_params=pltpu.CompilerParams(
          needs_layout_passes=False, use_tc_tiling_on_sc=False),
      scratch_types=[
          pltpu.VMEM((BW,), jnp.int32),
          pltpu.VMEM((BW,), jnp.int32),
          pltpu.VMEM((NCH, 128, D), jnp.float32),
          pltpu.VMEM((NCH, 128, D), jnp.float32),
          pltpu.VMEM((VP,), jnp.float32),
          pltpu.VMEM((VP,), jnp.float32),
          pltpu.VMEM((_L, _L), jnp.float32),
          pltpu.VMEM((BW,), jnp.float32),
          pltpu.SemaphoreType.DMA,
          pltpu.SemaphoreType.DMA,
      ],
  )
  def glove(center_hbm, context_hbm, cw_hbm, cb_hbm, xw_hbm, xb_hbm,
            out_hbm, idx_c, idx_x, rows_c, rows_x, cb_v, xb_v, tscr,
            out_v, sem_c, sem_x):
    wid = lax.axis_index("s") * _NC + lax.axis_index("c")
    base = wid * BW

    pltpu.sync_copy(center_hbm.at[pl.ds(base, BW)], idx_c)
    pltpu.sync_copy(context_hbm.at[pl.ds(base, BW)], idx_x)

    copies = []
    for j in range(NCH):
      copies.append(pltpu.async_copy(
          cw_hbm.at[idx_c.at[pl.ds(j * 128, 128)]], rows_c.at[j], sem_c))
      copies.append(pltpu.async_copy(
          xw_hbm.at[idx_x.at[pl.ds(j * 128, 128)]], rows_x.at[j], sem_x))
    pltpu.sync_copy(cb_hbm, cb_v)
    pltpu.sync_copy(xb_hbm, xb_v)
    for c in copies:
      c.wait()

    iot = lax.iota(jnp.int32, _L)

    for j in range(NCH):
      def group(g, _, j=j):
        for i in range(_L):
          row = g * _L + i
          s = (rows_c[j, row, pl.ds(0, _L)] * rows_x[j, row, pl.ds(0, _L)]
               + rows_c[j, row, pl.ds(_L, _L)] * rows_x[j, row, pl.ds(_L, _L)])
          for cc in range(2, D // _L):
            s = s + (rows_c[j, row, pl.ds(cc * _L, _L)]
                     * rows_x[j, row, pl.ds(cc * _L, _L)])
          tscr[i, :] = s
        ci = idx_c[pl.ds(j * 128 + g * _L, _L)]
        xi = idx_x[pl.ds(j * 128 + g * _L, _L)]
        acc = (plsc.load_gather(cb_v, [ci])
               + plsc.load_gather(xb_v, [xi]))
        for l in range(_L):
          col = plsc.load_gather(
              tscr, [iot, jnp.full((_L,), l, jnp.int32)])
          acc = acc + col
        out_v[pl.ds(j * 128 + g * _L, _L)] = acc
        return _

      lax.fori_loop(0, NG, group, 0)

    pltpu.sync_copy(out_v, out_hbm.at[pl.ds(base, BW)])

  return glove


@jax.jit
def kernel(center, context, center_weight, center_bias, context_weight,
           context_bias):
  B = center.shape[0]
  V, D = center_weight.shape
  VP = (V + 255) // 256 * 256
  cb = jnp.zeros((VP,), jnp.float32).at[:V].set(center_bias[:, 0])
  xb = jnp.zeros((VP,), jnp.float32).at[:V].set(context_bias[:, 0])
  glove = _make_glove_kernel(B, V, D, VP)
  return glove(center.astype(jnp.int32), context.astype(jnp.int32),
               center_weight, cb, context_weight, xb)
